# baseline (speedup 1.0000x reference)
import functools
import math

import jax
import jax.numpy as jnp
from jax import lax
from jax.experimental import pallas as pl
from jax.experimental.pallas import tpu as pltpu
from jax.experimental.pallas import tpu_sc as plsc

D_MODEL = 64
CHUNK = 128
LANES = 16


@functools.cache
def _build(B: int, V: int, D: int):
    info = plsc.get_sparse_core_info()
    nc, ns = info.num_cores, info.num_subcores
    nw = nc * ns
    b_per_w = B // nw
    n_chunks = b_per_w // CHUNK
    scale = math.sqrt(D)

    mesh = plsc.VectorSubcoreMesh(core_axis_name="c", subcore_axis_name="s")

    @functools.partial(
        pl.kernel,
        out_type=jax.ShapeDtypeStruct((B, D), jnp.float32),
        mesh=mesh,
        scratch_types=[
            pltpu.VMEM((n_chunks, CHUNK), jnp.int32),
            pltpu.VMEM((CHUNK, D), jnp.float32),
            pltpu.SemaphoreType.DMA,
        ],
        compiler_params=pltpu.CompilerParams(use_tc_tiling_on_sc=False),
    )
    def emb(idx_hbm, tbl_hbm, out_hbm, idx_v, rows_v, sem):
        wid = lax.axis_index("s") * nc + lax.axis_index("c")
        chunk0 = wid * n_chunks
        pltpu.sync_copy(idx_hbm.at[pl.ds(chunk0, n_chunks)], idx_v)

        def step(j, carry):
            pltpu.async_copy(tbl_hbm.at[idx_v.at[j]], rows_v, sem).wait()

            def scale_row(r, c2):
                for c in range(D // LANES):
                    sl = pl.ds(c * LANES, LANES)
                    rows_v[r, sl] = rows_v[r, sl] * scale
                return c2

            lax.fori_loop(0, CHUNK, scale_row, 0, unroll=4)
            pltpu.sync_copy(
                rows_v, out_hbm.at[pl.ds((chunk0 + j) * CHUNK, CHUNK)]
            )
            return carry

        lax.fori_loop(0, n_chunks, step, 0)

    return emb


def kernel(x, table):
    b0, b1 = x.shape
    B = b0 * b1
    V, D = table.shape
    idx = x.reshape(B // CHUNK, CHUNK).astype(jnp.int32)
    out = _build(B, V, D)(idx, table)
    return out.reshape(b0, b1, D)
